# baseline (speedup 1.0000x reference)
import functools

import jax
import jax.numpy as jnp
from jax import lax
from jax.experimental import pallas as pl
from jax.experimental.pallas import tpu as pltpu
from jax.experimental.pallas import tpu_sc as plsc

K = 8


@functools.lru_cache(maxsize=None)
def _build(b: int, l: int, d: int, vocab: int):
    info = plsc.get_sparse_core_info()
    nc, ns = info.num_cores, info.num_subcores
    nw = nc * ns
    assert b % (nw * 2 * K) == 0
    bpw = b // nw
    n_groups = bpw // K

    mesh = plsc.VectorSubcoreMesh(core_axis_name="c", subcore_axis_name="s")

    @functools.partial(
        pl.kernel,
        out_type=jax.ShapeDtypeStruct((b, l, d), jnp.float32),
        mesh=mesh,
        scratch_types=[
            pltpu.VMEM((bpw, l), jnp.int32),
            pltpu.VMEM((K, l, d), jnp.float32),
            pltpu.VMEM((K, l, d), jnp.float32),
            pltpu.SemaphoreType.DMA,
            pltpu.SemaphoreType.DMA,
        ],
        compiler_params=pltpu.CompilerParams(use_tc_tiling_on_sc=False),
    )
    def emb_kernel(x_hbm, emb_hbm, out_hbm, idx_v, rows_a, rows_b, sem_a,
                   sem_b):
        wid = lax.axis_index("s") * nc + lax.axis_index("c")
        pltpu.sync_copy(x_hbm.at[pl.ds(wid * bpw, bpw)], idx_v)
        row_base = wid * bpw
        bufs = ((rows_a, sem_a), (rows_b, sem_b))

        def issue(g, rows, sem):
            for j in range(K):
                pltpu.make_async_copy(
                    emb_hbm.at[idx_v.at[g * K + j]], rows.at[j], sem,
                ).start()

        def drain(g, rows, sem):
            for j in range(K):
                pltpu.make_async_copy(
                    emb_hbm.at[idx_v.at[g * K + j]], rows.at[j], sem,
                ).wait()

        issue(0, rows_a, sem_a)
        issue(1, rows_b, sem_b)

        def pair_body(p, carry):
            g = 2 * p
            for parity, (rows, sem) in enumerate(bufs):
                gg = g + parity
                drain(gg, rows, sem)
                pltpu.sync_copy(rows,
                                out_hbm.at[pl.ds(row_base + gg * K, K)])

                @pl.when(gg + 2 < n_groups)
                def _():
                    issue(gg + 2, rows, sem)

            return carry

        lax.fori_loop(0, n_groups // 2, pair_body, 0)

    return emb_kernel


def kernel(x, emb):
    b, l = x.shape
    return _build(b, l, emb.shape[1], emb.shape[0])(x.astype(jnp.int32), emb)
